# baseline (device time: 24458 ns/iter reference)
import os

import jax
import jax.numpy as jnp
from jax import lax
from jax.experimental import pallas as pl
from jax.experimental.pallas import tpu as pltpu

_SKIP_COMM = os.environ.get("SKIP_COMM", "0") == "1"

N_DEV = 4
B_LOC = 2
SQ = 256
SKV = 256
HQ = 16
DH = 64
D_MODEL = 512
D_HEADS = HQ * DH
CHUNK = D_HEADS // N_DEV
H_PER = HQ // N_DEV
HALF_Q = D_MODEL // 2
HALF_O = CHUNK // 2


def kernel(x, Wq, K_ext, V_ext, Wo):
    gb0 = lax.axis_index("i") * B_LOC
    k_loc = jnp.transpose(
        lax.dynamic_slice_in_dim(K_ext, gb0, B_LOC, axis=0),
        (0, 2, 1, 3)).astype(jnp.bfloat16)
    v_loc = jnp.transpose(
        lax.dynamic_slice_in_dim(V_ext, gb0, B_LOC, axis=0),
        (0, 2, 1, 3)).astype(jnp.bfloat16)
    x_bf = x.astype(jnp.bfloat16)

    def body(x_ref, wq_ref, k_ref, v_ref, wo_ref, out_ref,
             commq, commo, sendq, recvq, sendo, recvo):
        my = lax.axis_index("i")
        left = lax.rem(my + N_DEV - 1, N_DEV)
        right = lax.rem(my + 1, N_DEV)
        opp = lax.rem(my + 2, N_DEV)

        barrier = pltpu.get_barrier_semaphore()
        for nbr in (left, right):
            pl.semaphore_signal(barrier, inc=1, device_id=(nbr,),
                                device_id_type=pl.DeviceIdType.MESH)
        pl.semaphore_wait(barrier, 2)

        commq[0] = wq_ref[...].astype(jnp.bfloat16)
        commo[0] = wo_ref[...].astype(jnp.bfloat16)

        def copies(src, dst, dev, s_idx, r_idx):
            cq = pltpu.make_async_remote_copy(
                src_ref=src(commq), dst_ref=dst(commq),
                send_sem=sendq.at[s_idx], recv_sem=recvq.at[r_idx],
                device_id=(dev,), device_id_type=pl.DeviceIdType.MESH)
            co = pltpu.make_async_remote_copy(
                src_ref=src(commo), dst_ref=dst(commo),
                send_sem=sendo.at[s_idx], recv_sem=recvo.at[r_idx],
                device_id=(dev,), device_id_type=pl.DeviceIdType.MESH)
            return cq, co

        aL = copies(lambda c: c.at[0], lambda c: c.at[2], left, 0, 0)
        aR = copies(lambda c: c.at[0], lambda c: c.at[1], right, 1, 1)
        bR_q = pltpu.make_async_remote_copy(
            src_ref=commq.at[1, 0:HALF_Q], dst_ref=commq.at[3, 0:HALF_Q],
            send_sem=sendq.at[2], recv_sem=recvq.at[2],
            device_id=(right,), device_id_type=pl.DeviceIdType.MESH)
        bR_o = pltpu.make_async_remote_copy(
            src_ref=commo.at[1, 0:HALF_O], dst_ref=commo.at[3, 0:HALF_O],
            send_sem=sendo.at[2], recv_sem=recvo.at[2],
            device_id=(right,), device_id_type=pl.DeviceIdType.MESH)
        bL_q = pltpu.make_async_remote_copy(
            src_ref=commq.at[2, HALF_Q:D_MODEL],
            dst_ref=commq.at[3, HALF_Q:D_MODEL],
            send_sem=sendq.at[3], recv_sem=recvq.at[3],
            device_id=(left,), device_id_type=pl.DeviceIdType.MESH)
        bL_o = pltpu.make_async_remote_copy(
            src_ref=commo.at[2, HALF_O:CHUNK],
            dst_ref=commo.at[3, HALF_O:CHUNK],
            send_sem=sendo.at[3], recv_sem=recvo.at[3],
            device_id=(left,), device_id_type=pl.DeviceIdType.MESH)

        ri = lax.broadcasted_iota(jnp.int32, (SQ, SKV), 0) // 64
        ci = lax.broadcasted_iota(jnp.int32, (SQ, SKV), 1) // 64
        mask = (ri == ci) | (ci == 0) | (lax.rem(ri + ci, 3) == 0)

        xb = [x_ref[b] for b in range(B_LOC)]

        def compute_chunk(slot, origin):
            wq_c = commq[slot]
            wo_c = commo[slot]
            for b in range(B_LOC):
                qc = jnp.dot(xb[b], wq_c,
                             preferred_element_type=jnp.float32)
                ctx_cols = []
                for j in range(H_PER):
                    hg = origin * H_PER + j
                    qh = qc[:, j * DH:(j + 1) * DH].astype(jnp.bfloat16)
                    kh = k_ref[b, pl.ds(hg, 1)].reshape(SKV, DH)
                    vh = v_ref[b, pl.ds(hg, 1)].reshape(SKV, DH)
                    s = lax.dot_general(
                        qh, kh, (((1,), (1,)), ((), ())),
                        preferred_element_type=jnp.float32) * 0.125
                    w = jnp.where(mask, jnp.exp(s), 0.0)
                    recip = 1.0 / jnp.sum(w, axis=1, keepdims=True)
                    ctx_raw = jnp.dot(w.astype(jnp.bfloat16), vh,
                                      preferred_element_type=jnp.float32)
                    ctx_cols.append(ctx_raw * recip)
                ctx = jnp.concatenate(ctx_cols, axis=1).astype(jnp.bfloat16)
                acc = jnp.dot(ctx, wo_c,
                              preferred_element_type=jnp.float32)
                if slot == 0:
                    out_ref[b] = acc
                else:
                    out_ref[b] = out_ref[b] + acc

        if not _SKIP_COMM:
            for c in aL + aR:
                c.start()
        compute_chunk(0, my)
        if not _SKIP_COMM:
            aR[0].wait_recv()
            aR[1].wait_recv()
            bR_q.start()
            bR_o.start()
            aL[0].wait_recv()
            aL[1].wait_recv()
            bL_q.start()
            bL_o.start()
        compute_chunk(1, left)
        compute_chunk(2, right)
        if not _SKIP_COMM:
            for c in (bR_q, bR_o, bL_q, bL_o):
                c.wait_recv()
        compute_chunk(3, opp)
        if not _SKIP_COMM:
            for c in aL + aR + (bR_q, bR_o, bL_q, bL_o):
                c.wait_send()

    return pl.pallas_call(
        body,
        out_shape=jax.ShapeDtypeStruct((B_LOC, SQ, D_MODEL), jnp.float32),
        in_specs=[pl.BlockSpec(memory_space=pltpu.VMEM)] * 5,
        out_specs=pl.BlockSpec(memory_space=pltpu.VMEM),
        scratch_shapes=[
            pltpu.VMEM((N_DEV, D_MODEL, CHUNK), jnp.bfloat16),
            pltpu.VMEM((N_DEV, CHUNK, D_MODEL), jnp.bfloat16),
            pltpu.SemaphoreType.DMA((4,)),
            pltpu.SemaphoreType.DMA((4,)),
            pltpu.SemaphoreType.DMA((4,)),
            pltpu.SemaphoreType.DMA((4,)),
        ],
        compiler_params=pltpu.CompilerParams(collective_id=0),
    )(x_bf, Wq, k_loc, v_loc, Wo)
